# baseline (speedup 1.0000x reference)
import functools

import jax
import jax.numpy as jnp
from jax import lax
from jax.experimental import pallas as pl
from jax.experimental.pallas import tpu as pltpu
from jax.experimental.pallas import tpu_sc as plsc

B, N, DIMS = 8, 4096, 3
C_IN, C_OUT, K, P = 64, 128, 16, 1024
C_MID = 32
C_HALF = 64
C_CAT = 96
DM = 2
DW = 128

Q = 128
TT = 256
NROWS = B * P * K + B * P


def _elu(x):
    return jnp.where(x > 0, x, jnp.exp(x) - 1.0)


def _col_from_row(row, n):
    r2 = jnp.broadcast_to(row, (n, n))
    ii = lax.broadcasted_iota(jnp.int32, (n, n), 0)
    jj = lax.broadcasted_iota(jnp.int32, (n, n), 1)
    return jnp.sum(jnp.where(ii == jj, r2, jnp.zeros_like(r2)), axis=1,
                   keepdims=True)


def _fps_body(px_ref, py_ref, pz_ref, rx_ref, ry_ref, rz_ref, cg_ref):
    px = px_ref[...]
    py = py_ref[...]
    pz = pz_ref[...]
    iota_n = lax.broadcasted_iota(jnp.int32, (B, N), 1)
    iota_p = lax.broadcasted_iota(jnp.int32, (B, P), 1)
    boff = lax.broadcasted_iota(jnp.int32, (B, 1), 0) * N

    def body(i, carry):
        dist, far, rx, ry, rz, cg = carry
        m = (iota_n == far).astype(px.dtype)
        cx = jnp.sum(px * m, axis=1, keepdims=True)
        cy = jnp.sum(py * m, axis=1, keepdims=True)
        cz = jnp.sum(pz * m, axis=1, keepdims=True)
        ohp = (iota_p == i)
        rx = rx + jnp.where(ohp, cx, 0.0)
        ry = ry + jnp.where(ohp, cy, 0.0)
        rz = rz + jnp.where(ohp, cz, 0.0)
        cg = cg + jnp.where(ohp, far + boff, 0)
        dx = px - cx
        dy = py - cy
        dz = pz - cz
        d = (dx * dx + dy * dy) + dz * dz
        dist = jnp.minimum(dist, d)
        mx = jnp.max(dist, axis=1, keepdims=True)
        far = jnp.min(jnp.where(dist == mx, iota_n, N), axis=1, keepdims=True)
        return dist, far, rx, ry, rz, cg

    dist0 = jnp.full((B, N), 1e10, jnp.float32)
    far0 = jnp.zeros((B, 1), jnp.int32)
    z = jnp.zeros((B, P), jnp.float32)
    zi = jnp.zeros((B, P), jnp.int32)
    _, _, rx, ry, rz, cg = lax.fori_loop(0, P, body,
                                         (dist0, far0, z, z, z, zi))
    rx_ref[...] = rx
    ry_ref[...] = ry
    rz_ref[...] = rz
    cg_ref[...] = cg


def _fps_call(px, py, pz):
    out = [jax.ShapeDtypeStruct((B, P), jnp.float32)] * 3 + [
        jax.ShapeDtypeStruct((B, P), jnp.int32)]
    return pl.pallas_call(_fps_body, out_shape=tuple(out))(px, py, pz)


def _table_body(pts_ref, fts_ref, w_ref, b_ref, out_ref):
    t = _elu(jnp.dot(fts_ref[...], w_ref[...],
                     preferred_element_type=jnp.float32) + b_ref[...])
    pad = jnp.zeros((pts_ref.shape[0], DW - DIMS - C_HALF), jnp.float32)
    out_ref[...] = jnp.concatenate([pts_ref[...], t, pad], axis=1)


def _table_call(pts_f, fts_f, w, b):
    tb = 2048
    grid = (B * N // tb,)
    return pl.pallas_call(
        _table_body,
        grid=grid,
        in_specs=[
            pl.BlockSpec((tb, DIMS), lambda i: (i, 0)),
            pl.BlockSpec((tb, C_IN), lambda i: (i, 0)),
            pl.BlockSpec((C_IN, C_HALF), lambda i: (0, 0)),
            pl.BlockSpec((1, C_HALF), lambda i: (0, 0)),
        ],
        out_specs=pl.BlockSpec((tb, DW), lambda i: (i, 0)),
        out_shape=jax.ShapeDtypeStruct((B * N, DW), jnp.float32),
    )(pts_f, fts_f, w, b)


def _knn_body(rx_ref, ry_ref, rz_ref, px_ref, py_ref, pz_ref, out_ref):
    b = pl.program_id(0)
    qx = _col_from_row(rx_ref[...].reshape(1, Q), Q)
    qy = _col_from_row(ry_ref[...].reshape(1, Q), Q)
    qz = _col_from_row(rz_ref[...].reshape(1, Q), Q)
    dx = qx - px_ref[...].reshape(1, N)
    dy = qy - py_ref[...].reshape(1, N)
    dz = qz - pz_ref[...].reshape(1, N)
    d2 = (dx * dx + dy * dy) + dz * dz
    iota_n = lax.broadcasted_iota(jnp.int32, (Q, N), 1)
    cols = []
    for _ in range(K):
        m = jnp.min(d2, axis=1, keepdims=True)
        sel = jnp.min(jnp.where(d2 == m, iota_n, N), axis=1, keepdims=True)
        cols.append(sel)
        d2 = jnp.where(iota_n == sel, jnp.inf, d2)
    idx = jnp.concatenate(cols, axis=1) + b * N
    out_ref[0] = idx


def _knn_call(rx, ry, rz, px, py, pz):
    grid = (B, P // Q)
    nq = P // Q
    r3 = lambda a: a.reshape(B * nq, 1, Q)
    p3 = lambda a: a.reshape(B, 1, N)
    rspec = pl.BlockSpec((1, 1, Q), lambda b, q: (b * nq + q, 0, 0))
    pspec = pl.BlockSpec((1, 1, N), lambda b, q: (b, 0, 0))
    return pl.pallas_call(
        _knn_body,
        grid=grid,
        in_specs=[rspec, rspec, rspec, pspec, pspec, pspec],
        out_specs=pl.BlockSpec((1, Q, K), lambda b, q: (b, q, 0)),
        out_shape=jax.ShapeDtypeStruct((B, P, K), jnp.int32),
    )(r3(rx), r3(ry), r3(rz), p3(px), p3(py), p3(pz))


NW = 32
RPW = NROWS // NW
CH = 128
NCH = RPW // CH


def _sc_gather_body(table_hbm, idx_hbm, out_hbm, idx_v, buf, sem):
    wid = lax.axis_index("s") * 2 + lax.axis_index("c")
    base = wid * RPW
    pltpu.sync_copy(idx_hbm.at[pl.ds(base, RPW)], idx_v)

    def chunk(g, carry):
        off = g * CH
        pltpu.async_copy(table_hbm.at[idx_v.at[pl.ds(off, CH)]], buf,
                         sem).wait()
        pltpu.sync_copy(buf, out_hbm.at[pl.ds(base + off, CH)])
        return carry

    lax.fori_loop(0, NCH, chunk, 0)


def _sc_gather_call(table, idx_all):
    mesh = plsc.VectorSubcoreMesh(core_axis_name="c", subcore_axis_name="s")
    f = pl.kernel(
        _sc_gather_body,
        out_type=jax.ShapeDtypeStruct((NROWS, DW), jnp.float32),
        mesh=mesh,
        scratch_types=[
            pltpu.VMEM((RPW,), jnp.int32),
            pltpu.VMEM((CH, DW), jnp.float32),
            pltpu.SemaphoreType.DMA,
        ],
    )
    return f(table, idx_all)


def _tail_body(g3_ref, rg_ref, wd1_ref, bd1_ref, wd2_ref, bd2_ref,
               wc3t_ref, bc3t_ref, wdc1_ref, bdc1t_ref, wdc2t_ref,
               bdc2tt_ref, wsdt_ref, bsd_ref, wsp_ref, bsp_ref, out_ref):
    g3 = g3_ref[...]
    rep3 = rg_ref[:, 0:DIMS].reshape(TT, 1, DIMS)
    pl3 = g3[:, :, 0:DIMS] - rep3

    plf = pl3.reshape(TT * K, DIMS)
    l1 = jnp.broadcast_to(bd1_ref[...], (TT * K, C_MID))
    for c in range(DIMS):
        l1 = l1 + plf[:, c:c + 1] * wd1_ref[c:c + 1, :]
    l1 = _elu(l1)
    l2 = _elu(jnp.dot(l1, wd2_ref[...],
                      preferred_element_type=jnp.float32) + bd2_ref[...])
    fc3 = jnp.concatenate([l2.reshape(TT, K, C_MID), g3[:, :, DIMS:DIMS + C_HALF]],
                          axis=2)

    x3t = jnp.broadcast_to(bc3t_ref[...], (TT, K, K))
    for k in range(K):
        for c in range(DIMS):
            w = wc3t_ref[k * DIMS + c]
            x3t = x3t + pl3[:, k, c:c + 1].reshape(TT, 1, 1) * w[None]
    x3t = _elu(x3t)

    cols = []
    for g in range(K):
        t = x3t[:, g:g + 1, :] * wdc1_ref[g][None]
        cols.append(jnp.sum(t, axis=2, keepdims=True))
    x4t = _elu(jnp.concatenate(cols, axis=2) + bdc1t_ref[...])

    x5tt = jnp.broadcast_to(bdc2tt_ref[...], (TT, K, K))
    for k2 in range(K):
        x5tt = x5tt + x4t[:, :, k2:k2 + 1] * wdc2t_ref[k2][None]

    fx = x5tt[:, :, 0:1] * fc3[:, 0:1, :]
    for j in range(1, K):
        fx = fx + x5tt[:, :, j:j + 1] * fc3[:, j:j + 1, :]

    dws = []
    for m in range(DM):
        t = fx * wsdt_ref[m][None]
        dws.append(jnp.sum(t, axis=1, keepdims=True).reshape(TT, C_CAT))
    dwb = jnp.concatenate(dws, axis=1) + bsd_ref[...]
    out = _elu(jnp.dot(dwb, wsp_ref[...],
                       preferred_element_type=jnp.float32) + bsp_ref[...])
    out_ref[...] = out


def _tail_call(g3, rg, wd1, bd1, wd2, bd2, wc3t, bc3t, wdc1, bdc1t, wdc2t,
               bdc2tt, wsdt, bsd_r, wsp_r, bsp):
    grid = (B * P // TT,)
    full = lambda a: pl.BlockSpec(a.shape, lambda i: (0,) * a.ndim)
    return pl.pallas_call(
        _tail_body,
        grid=grid,
        in_specs=[
            pl.BlockSpec((TT, K, DW), lambda i: (i, 0, 0)),
            pl.BlockSpec((TT, DW), lambda i: (i, 0)),
            full(wd1), full(bd1), full(wd2), full(bd2),
            full(wc3t), full(bc3t), full(wdc1), full(bdc1t),
            full(wdc2t), full(bdc2tt), full(wsdt), full(bsd_r),
            full(wsp_r), full(bsp),
        ],
        out_specs=pl.BlockSpec((TT, C_OUT), lambda i: (i, 0)),
        out_shape=jax.ShapeDtypeStruct((B * P, C_OUT), jnp.float32),
    )(g3, rg, wd1, bd1, wd2, bd2, wc3t, bc3t, wdc1, bdc1t, wdc2t, bdc2tt,
      wsdt, bsd_r, wsp_r, bsp)


def kernel(pts, fts, W_dense, b_dense, W_d1, b_d1, W_d2, b_d2, W_conv,
           b_conv, W_dc1, b_dc1, W_dc2, b_dc2, W_sd, b_sd, W_sp, b_sp):
    px = pts[:, :, 0]
    py = pts[:, :, 1]
    pz = pts[:, :, 2]

    rx, ry, rz, cg = _fps_call(px, py, pz)

    table = _table_call(pts.reshape(B * N, DIMS), fts.reshape(B * N, C_IN),
                        W_dense, b_dense.reshape(1, C_HALF))

    idx3 = _knn_call(rx, ry, rz, px, py, pz)
    idx_all = jnp.concatenate([idx3.reshape(-1), cg.reshape(-1)])

    rows = _sc_gather_call(table, idx_all)
    g3 = rows[:B * P * K].reshape(B * P, K, DW)
    rg = rows[B * P * K:]

    wc3t = W_conv.reshape(K, K, DIMS, K).transpose(3, 2, 1, 0).reshape(
        K * DIMS, K, K)
    bc3t = b_conv.reshape(K, K).T[None]
    bdc1t = b_dc1.reshape(K, K).T[None]
    wdc2t = W_dc2.transpose(2, 0, 1)
    bdc2tt = b_dc2.reshape(K, K)[None]
    wsdt = W_sd.transpose(1, 2, 0)
    bsd_r = b_sd.reshape(C_CAT, DM).T.reshape(1, DM * C_CAT)
    wsp_r = W_sp.reshape(C_CAT, DM, C_OUT).transpose(1, 0, 2).reshape(
        DM * C_CAT, C_OUT)

    fts_p = _tail_call(g3, rg, W_d1, b_d1.reshape(1, C_MID), W_d2,
                       b_d2.reshape(1, C_MID), wc3t, bc3t, W_dc1, bdc1t,
                       wdc2t, bdc2tt, wsdt, bsd_r, wsp_r,
                       b_sp.reshape(1, C_OUT))

    rep_pts = jnp.stack([rx, ry, rz], axis=-1)
    return rep_pts, fts_p.reshape(B, P, C_OUT)


# speedup vs baseline: 1.7389x; 1.7389x over previous
import functools

import jax
import jax.numpy as jnp
from jax import lax
from jax.experimental import pallas as pl
from jax.experimental.pallas import tpu as pltpu
from jax.experimental.pallas import tpu_sc as plsc

B, N, DIMS = 8, 4096, 3
C_IN, C_OUT, K, P = 64, 128, 16, 1024
C_MID = 32
C_HALF = 64
C_CAT = 96
DM = 2
DW = 128

Q = 128
TT = 256
NROWS = B * P * K + B * P


def _elu(x):
    return jnp.where(x > 0, x, jnp.exp(x) - 1.0)


def _col_from_row(row, n):
    r2 = jnp.broadcast_to(row, (n, n))
    ii = lax.broadcasted_iota(jnp.int32, (n, n), 0)
    jj = lax.broadcasted_iota(jnp.int32, (n, n), 1)
    return jnp.sum(jnp.where(ii == jj, r2, jnp.zeros_like(r2)), axis=1,
                   keepdims=True)


def _fps_body(px_ref, py_ref, pz_ref, rx_ref, ry_ref, rz_ref, cg_ref):
    px = px_ref[...]
    py = py_ref[...]
    pz = pz_ref[...]
    iota_n = lax.broadcasted_iota(jnp.int32, (B, N), 1)
    iota_p = lax.broadcasted_iota(jnp.int32, (B, P), 1)
    boff = lax.broadcasted_iota(jnp.int32, (B, 1), 0) * N

    def body(i, carry):
        dist, far, rx, ry, rz, cg = carry
        m = (iota_n == far).astype(px.dtype)
        cx = jnp.sum(px * m, axis=1, keepdims=True)
        cy = jnp.sum(py * m, axis=1, keepdims=True)
        cz = jnp.sum(pz * m, axis=1, keepdims=True)
        ohp = (iota_p == i)
        rx = rx + jnp.where(ohp, cx, 0.0)
        ry = ry + jnp.where(ohp, cy, 0.0)
        rz = rz + jnp.where(ohp, cz, 0.0)
        cg = cg + jnp.where(ohp, far + boff, 0)
        dx = px - cx
        dy = py - cy
        dz = pz - cz
        d = (dx * dx + dy * dy) + dz * dz
        dist = jnp.minimum(dist, d)
        mx = jnp.max(dist, axis=1, keepdims=True)
        far = jnp.min(jnp.where(dist == mx, iota_n, N), axis=1, keepdims=True)
        return dist, far, rx, ry, rz, cg

    dist0 = jnp.full((B, N), 1e10, jnp.float32)
    far0 = jnp.zeros((B, 1), jnp.int32)
    z = jnp.zeros((B, P), jnp.float32)
    zi = jnp.zeros((B, P), jnp.int32)
    _, _, rx, ry, rz, cg = lax.fori_loop(0, P, body,
                                         (dist0, far0, z, z, z, zi))
    rx_ref[...] = rx
    ry_ref[...] = ry
    rz_ref[...] = rz
    cg_ref[...] = cg


def _fps_call(px, py, pz):
    out = [jax.ShapeDtypeStruct((B, P), jnp.float32)] * 3 + [
        jax.ShapeDtypeStruct((B, P), jnp.int32)]
    return pl.pallas_call(_fps_body, out_shape=tuple(out))(px, py, pz)


def _table_body(pts_ref, fts_ref, w_ref, b_ref, out_ref):
    t = _elu(jnp.dot(fts_ref[...], w_ref[...],
                     preferred_element_type=jnp.float32) + b_ref[...])
    pad = jnp.zeros((pts_ref.shape[0], DW - DIMS - C_HALF), jnp.float32)
    out_ref[...] = jnp.concatenate([pts_ref[...], t, pad], axis=1)


def _table_call(pts_f, fts_f, w, b):
    tb = 2048
    grid = (B * N // tb,)
    return pl.pallas_call(
        _table_body,
        grid=grid,
        in_specs=[
            pl.BlockSpec((tb, DIMS), lambda i: (i, 0)),
            pl.BlockSpec((tb, C_IN), lambda i: (i, 0)),
            pl.BlockSpec((C_IN, C_HALF), lambda i: (0, 0)),
            pl.BlockSpec((1, C_HALF), lambda i: (0, 0)),
        ],
        out_specs=pl.BlockSpec((tb, DW), lambda i: (i, 0)),
        out_shape=jax.ShapeDtypeStruct((B * N, DW), jnp.float32),
    )(pts_f, fts_f, w, b)


def _knn_body(rx_ref, ry_ref, rz_ref, px_ref, py_ref, pz_ref, out_ref):
    b = pl.program_id(0)
    qx = _col_from_row(rx_ref[...].reshape(1, Q), Q)
    qy = _col_from_row(ry_ref[...].reshape(1, Q), Q)
    qz = _col_from_row(rz_ref[...].reshape(1, Q), Q)
    dx = qx - px_ref[...].reshape(1, N)
    dy = qy - py_ref[...].reshape(1, N)
    dz = qz - pz_ref[...].reshape(1, N)
    d2 = (dx * dx + dy * dy) + dz * dz
    iota_n = lax.broadcasted_iota(jnp.int32, (Q, N), 1)
    cols = []
    for _ in range(K):
        m = jnp.min(d2, axis=1, keepdims=True)
        sel = jnp.min(jnp.where(d2 == m, iota_n, N), axis=1, keepdims=True)
        cols.append(sel)
        d2 = jnp.where(iota_n == sel, jnp.inf, d2)
    idx = jnp.concatenate(cols, axis=1) + b * N
    out_ref[0] = idx


def _knn_call(rx, ry, rz, px, py, pz):
    grid = (B, P // Q)
    nq = P // Q
    r3 = lambda a: a.reshape(B * nq, 1, Q)
    p3 = lambda a: a.reshape(B, 1, N)
    rspec = pl.BlockSpec((1, 1, Q), lambda b, q: (b * nq + q, 0, 0))
    pspec = pl.BlockSpec((1, 1, N), lambda b, q: (b, 0, 0))
    return pl.pallas_call(
        _knn_body,
        grid=grid,
        in_specs=[rspec, rspec, rspec, pspec, pspec, pspec],
        out_specs=pl.BlockSpec((1, Q, K), lambda b, q: (b, q, 0)),
        out_shape=jax.ShapeDtypeStruct((B, P, K), jnp.int32),
    )(r3(rx), r3(ry), r3(rz), p3(px), p3(py), p3(pz))


NW = 32
RPW = NROWS // NW
CH = 128
NCH = RPW // CH


def _sc_gather_body(table_hbm, idx_hbm, out_hbm, idx_v, buf, sem):
    wid = lax.axis_index("s") * 2 + lax.axis_index("c")
    base = wid * RPW
    pltpu.sync_copy(idx_hbm.at[pl.ds(base, RPW)], idx_v)

    def chunk(g, carry):
        off = g * CH
        pltpu.async_copy(table_hbm.at[idx_v.at[pl.ds(off, CH)]], buf,
                         sem).wait()
        pltpu.sync_copy(buf, out_hbm.at[pl.ds(base + off, CH)])
        return carry

    lax.fori_loop(0, NCH, chunk, 0)


def _sc_gather_call(table, idx_all):
    mesh = plsc.VectorSubcoreMesh(core_axis_name="c", subcore_axis_name="s")
    f = pl.kernel(
        _sc_gather_body,
        out_type=jax.ShapeDtypeStruct((NROWS, DW), jnp.float32),
        mesh=mesh,
        scratch_types=[
            pltpu.VMEM((RPW,), jnp.int32),
            pltpu.VMEM((CH, DW), jnp.float32),
            pltpu.SemaphoreType.DMA,
        ],
    )
    return f(table, idx_all)


def _tail_body(g3_ref, g2_ref, rg_ref, wd1_ref, bd1_ref, wd2_ref, bd2_ref,
               wc48_ref, bc_ref, wdc1b_ref, bdc1_ref, wdc2b_ref,
               bdc2_ref, wsdt_ref, bsd_ref, wsp_ref, bsp_ref, out_ref):
    g3 = g3_ref[...]
    g2 = g2_ref[...]
    rep3 = rg_ref[:, 0:DIMS].reshape(TT, 1, DIMS)
    pl3 = g3[:, :, 0:DIMS] - rep3

    plf = pl3.reshape(TT * K, DIMS)
    l1 = jnp.broadcast_to(bd1_ref[...], (TT * K, C_MID))
    for c in range(DIMS):
        l1 = l1 + plf[:, c:c + 1] * wd1_ref[c:c + 1, :]
    l1 = _elu(l1)
    l2 = _elu(jnp.dot(l1, wd2_ref[...],
                      preferred_element_type=jnp.float32) + bd2_ref[...])
    fc3 = jnp.concatenate([l2.reshape(TT, K, C_MID), g3[:, :, DIMS:DIMS + C_HALF]],
                          axis=2)

    rgc = rg_ref[:, 0:DIMS]
    pl48 = jnp.concatenate(
        [g2[:, k * DW:k * DW + DIMS] for k in range(K)], axis=1)
    pl48 = pl48 - jnp.concatenate([rgc] * K, axis=1)

    x0 = _elu(jnp.dot(pl48, wc48_ref[...],
                      preferred_element_type=jnp.float32) + bc_ref[...])
    x1 = _elu(jnp.dot(x0, wdc1b_ref[...],
                      preferred_element_type=jnp.float32) + bdc1_ref[...])
    x2 = jnp.dot(x1, wdc2b_ref[...],
                 preferred_element_type=jnp.float32) + bdc2_ref[...]
    x5tt = x2.reshape(TT, K, K)

    fx = x5tt[:, :, 0:1] * fc3[:, 0:1, :]
    for j in range(1, K):
        fx = fx + x5tt[:, :, j:j + 1] * fc3[:, j:j + 1, :]

    dws = []
    for m in range(DM):
        t = fx * wsdt_ref[m][None]
        dws.append(jnp.sum(t, axis=1, keepdims=True).reshape(TT, C_CAT))
    dwb = jnp.concatenate(dws, axis=1) + bsd_ref[...]
    out = _elu(jnp.dot(dwb, wsp_ref[...],
                       preferred_element_type=jnp.float32) + bsp_ref[...])
    out_ref[...] = out


def _tail_call(g3, g2, rg, wd1, bd1, wd2, bd2, wc48, bc, wdc1b, bdc1,
               wdc2b, bdc2, wsdt, bsd_r, wsp_r, bsp):
    grid = (B * P // TT,)
    full = lambda a: pl.BlockSpec(a.shape, lambda i: (0,) * a.ndim)
    return pl.pallas_call(
        _tail_body,
        grid=grid,
        in_specs=[
            pl.BlockSpec((TT, K, DW), lambda i: (i, 0, 0)),
            pl.BlockSpec((TT, K * DW), lambda i: (i, 0)),
            pl.BlockSpec((TT, DW), lambda i: (i, 0)),
            full(wd1), full(bd1), full(wd2), full(bd2),
            full(wc48), full(bc), full(wdc1b), full(bdc1),
            full(wdc2b), full(bdc2), full(wsdt), full(bsd_r),
            full(wsp_r), full(bsp),
        ],
        out_specs=pl.BlockSpec((TT, C_OUT), lambda i: (i, 0)),
        out_shape=jax.ShapeDtypeStruct((B * P, C_OUT), jnp.float32),
    )(g3, g2, rg, wd1, bd1, wd2, bd2, wc48, bc, wdc1b, bdc1, wdc2b, bdc2,
      wsdt, bsd_r, wsp_r, bsp)


def kernel(pts, fts, W_dense, b_dense, W_d1, b_d1, W_d2, b_d2, W_conv,
           b_conv, W_dc1, b_dc1, W_dc2, b_dc2, W_sd, b_sd, W_sp, b_sp):
    px = pts[:, :, 0]
    py = pts[:, :, 1]
    pz = pts[:, :, 2]

    rx, ry, rz, cg = _fps_call(px, py, pz)

    table = _table_call(pts.reshape(B * N, DIMS), fts.reshape(B * N, C_IN),
                        W_dense, b_dense.reshape(1, C_HALF))

    idx3 = _knn_call(rx, ry, rz, px, py, pz)
    idx_all = jnp.concatenate([idx3.reshape(-1), cg.reshape(-1)])

    rows = _sc_gather_call(table, idx_all)
    g3 = rows[:B * P * K].reshape(B * P, K, DW)
    g2 = rows[:B * P * K].reshape(B * P, K * DW)
    rg = rows[B * P * K:]

    wc48 = W_conv.transpose(2, 1, 0).reshape(K * DIMS, K * K)
    delta = jnp.eye(K, dtype=jnp.float32)
    wdc1b = jnp.einsum('gmk,hg->khgm', W_dc1, delta).reshape(K * K, K * K)
    wdc2b = jnp.einsum('gmk,hg->khgm', W_dc2, delta).reshape(K * K, K * K)
    wsdt = W_sd.transpose(1, 2, 0)
    bsd_r = b_sd.reshape(C_CAT, DM).T.reshape(1, DM * C_CAT)
    wsp_r = W_sp.reshape(C_CAT, DM, C_OUT).transpose(1, 0, 2).reshape(
        DM * C_CAT, C_OUT)

    fts_p = _tail_call(g3, g2, rg, W_d1, b_d1.reshape(1, C_MID), W_d2,
                       b_d2.reshape(1, C_MID), wc48, b_conv.reshape(1, K * K),
                       wdc1b, b_dc1.reshape(1, K * K),
                       wdc2b, b_dc2.reshape(1, K * K), wsdt, bsd_r, wsp_r,
                       b_sp.reshape(1, C_OUT))

    rep_pts = jnp.stack([rx, ry, rz], axis=-1)
    return rep_pts, fts_p.reshape(B, P, C_OUT)


# speedup vs baseline: 1.8324x; 1.0538x over previous
import functools

import jax
import jax.numpy as jnp
from jax import lax
from jax.experimental import pallas as pl
from jax.experimental.pallas import tpu as pltpu
from jax.experimental.pallas import tpu_sc as plsc

B, N, DIMS = 8, 4096, 3
C_IN, C_OUT, K, P = 64, 128, 16, 1024
C_MID = 32
C_HALF = 64
C_CAT = 96
DM = 2
DW = 128

Q = 256
TT = 256
NROWS = B * P * K + B * P


def _elu(x):
    return jnp.where(x > 0, x, jnp.exp(x) - 1.0)


def _col_from_row(row, n):
    r2 = jnp.broadcast_to(row, (n, n))
    ii = lax.broadcasted_iota(jnp.int32, (n, n), 0)
    jj = lax.broadcasted_iota(jnp.int32, (n, n), 1)
    return jnp.sum(jnp.where(ii == jj, r2, jnp.zeros_like(r2)), axis=1,
                   keepdims=True)


def _fps_body(px_ref, py_ref, pz_ref, rx_ref, ry_ref, rz_ref, cg_ref):
    px = px_ref[...]
    py = py_ref[...]
    pz = pz_ref[...]
    iota_n = lax.broadcasted_iota(jnp.int32, (B, N), 1)
    iota_p = lax.broadcasted_iota(jnp.int32, (B, P), 1)
    boff = lax.broadcasted_iota(jnp.int32, (B, 1), 0) * N
    ninf = jnp.float32(-jnp.inf)

    def body(i, carry):
        dist, far, cx, cy, cz, rx, ry, rz, cg = carry
        ohp = (iota_p == i)
        rx = rx + jnp.where(ohp, cx, 0.0)
        ry = ry + jnp.where(ohp, cy, 0.0)
        rz = rz + jnp.where(ohp, cz, 0.0)
        cg = cg + jnp.where(ohp, far + boff, 0)
        dx = px - cx
        dy = py - cy
        dz = pz - cz
        d = (dx * dx + dy * dy) + dz * dz
        dist = jnp.minimum(dist, d)
        mx = jnp.max(dist, axis=1, keepdims=True)
        sel = dist == mx
        far = jnp.min(jnp.where(sel, iota_n, N), axis=1, keepdims=True)
        cx = jnp.max(jnp.where(sel, px, ninf), axis=1, keepdims=True)
        cy = jnp.max(jnp.where(sel, py, ninf), axis=1, keepdims=True)
        cz = jnp.max(jnp.where(sel, pz, ninf), axis=1, keepdims=True)
        return dist, far, cx, cy, cz, rx, ry, rz, cg

    dist0 = jnp.full((B, N), 1e10, jnp.float32)
    far0 = jnp.zeros((B, 1), jnp.int32)
    z = jnp.zeros((B, P), jnp.float32)
    zi = jnp.zeros((B, P), jnp.int32)
    st = lax.fori_loop(0, P, body,
                       (dist0, far0, px[:, 0:1], py[:, 0:1], pz[:, 0:1],
                        z, z, z, zi))
    _, _, _, _, _, rx, ry, rz, cg = st
    rx_ref[...] = rx
    ry_ref[...] = ry
    rz_ref[...] = rz
    cg_ref[...] = cg


def _fps_call(px, py, pz):
    out = [jax.ShapeDtypeStruct((B, P), jnp.float32)] * 3 + [
        jax.ShapeDtypeStruct((B, P), jnp.int32)]
    return pl.pallas_call(_fps_body, out_shape=tuple(out))(px, py, pz)


def _table_body(pts_ref, fts_ref, w_ref, b_ref, out_ref):
    t = _elu(jnp.dot(fts_ref[...], w_ref[...],
                     preferred_element_type=jnp.float32) + b_ref[...])
    pad = jnp.zeros((pts_ref.shape[0], DW - DIMS - C_HALF), jnp.float32)
    out_ref[...] = jnp.concatenate([pts_ref[...], t, pad], axis=1)


def _table_call(pts_f, fts_f, w, b):
    tb = 2048
    grid = (B * N // tb,)
    return pl.pallas_call(
        _table_body,
        grid=grid,
        in_specs=[
            pl.BlockSpec((tb, DIMS), lambda i: (i, 0)),
            pl.BlockSpec((tb, C_IN), lambda i: (i, 0)),
            pl.BlockSpec((C_IN, C_HALF), lambda i: (0, 0)),
            pl.BlockSpec((1, C_HALF), lambda i: (0, 0)),
        ],
        out_specs=pl.BlockSpec((tb, DW), lambda i: (i, 0)),
        out_shape=jax.ShapeDtypeStruct((B * N, DW), jnp.float32),
    )(pts_f, fts_f, w, b)


def _knn_body(rx_ref, ry_ref, rz_ref, px_ref, py_ref, pz_ref, out_ref):
    b = pl.program_id(0)
    qx = _col_from_row(rx_ref[...].reshape(1, Q), Q)
    qy = _col_from_row(ry_ref[...].reshape(1, Q), Q)
    qz = _col_from_row(rz_ref[...].reshape(1, Q), Q)
    dx = qx - px_ref[...].reshape(1, N)
    dy = qy - py_ref[...].reshape(1, N)
    dz = qz - pz_ref[...].reshape(1, N)
    d2 = (dx * dx + dy * dy) + dz * dz
    iota_n = lax.broadcasted_iota(jnp.int32, (Q, N), 1)
    cols = []
    for _ in range(K):
        m = jnp.min(d2, axis=1, keepdims=True)
        sel = jnp.min(jnp.where(d2 == m, iota_n, N), axis=1, keepdims=True)
        cols.append(sel)
        d2 = jnp.where(iota_n == sel, jnp.inf, d2)
    idx = jnp.concatenate(cols, axis=1) + b * N
    out_ref[0] = idx


def _knn_call(rx, ry, rz, px, py, pz):
    grid = (B, P // Q)
    nq = P // Q
    r3 = lambda a: a.reshape(B * nq, 1, Q)
    p3 = lambda a: a.reshape(B, 1, N)
    rspec = pl.BlockSpec((1, 1, Q), lambda b, q: (b * nq + q, 0, 0))
    pspec = pl.BlockSpec((1, 1, N), lambda b, q: (b, 0, 0))
    return pl.pallas_call(
        _knn_body,
        grid=grid,
        in_specs=[rspec, rspec, rspec, pspec, pspec, pspec],
        out_specs=pl.BlockSpec((1, Q, K), lambda b, q: (b, q, 0)),
        out_shape=jax.ShapeDtypeStruct((B, P, K), jnp.int32),
    )(r3(rx), r3(ry), r3(rz), p3(px), p3(py), p3(pz))


NW = 32
RPW = NROWS // NW
CH = 128
NCH = RPW // CH


def _sc_gather_body(table_hbm, idx_hbm, out_hbm, idx_v, buf, sem):
    wid = lax.axis_index("s") * 2 + lax.axis_index("c")
    base = wid * RPW
    pltpu.sync_copy(idx_hbm.at[pl.ds(base, RPW)], idx_v)

    def chunk(g, carry):
        off = g * CH
        pltpu.async_copy(table_hbm.at[idx_v.at[pl.ds(off, CH)]], buf,
                         sem).wait()
        pltpu.sync_copy(buf, out_hbm.at[pl.ds(base + off, CH)])
        return carry

    lax.fori_loop(0, NCH, chunk, 0)


def _sc_gather_call(table, idx_all):
    mesh = plsc.VectorSubcoreMesh(core_axis_name="c", subcore_axis_name="s")
    f = pl.kernel(
        _sc_gather_body,
        out_type=jax.ShapeDtypeStruct((NROWS, DW), jnp.float32),
        mesh=mesh,
        scratch_types=[
            pltpu.VMEM((RPW,), jnp.int32),
            pltpu.VMEM((CH, DW), jnp.float32),
            pltpu.SemaphoreType.DMA,
        ],
    )
    return f(table, idx_all)


def _tail_body(g3_ref, g2_ref, rg_ref, wd1_ref, bd1_ref, wd2_ref, bd2_ref,
               wc48_ref, bc_ref, wdc1b_ref, bdc1_ref, wdc2b_ref,
               bdc2_ref, wsdt_ref, bsd_ref, wsp_ref, bsp_ref, out_ref):
    g3 = g3_ref[...]
    g2 = g2_ref[...]
    rep3 = rg_ref[:, 0:DIMS].reshape(TT, 1, DIMS)
    pl3 = g3[:, :, 0:DIMS] - rep3

    plf = pl3.reshape(TT * K, DIMS)
    l1 = jnp.broadcast_to(bd1_ref[...], (TT * K, C_MID))
    for c in range(DIMS):
        l1 = l1 + plf[:, c:c + 1] * wd1_ref[c:c + 1, :]
    l1 = _elu(l1)
    l2 = _elu(jnp.dot(l1, wd2_ref[...],
                      preferred_element_type=jnp.float32) + bd2_ref[...])
    fc3 = jnp.concatenate([l2.reshape(TT, K, C_MID), g3[:, :, DIMS:DIMS + C_HALF]],
                          axis=2)

    rgc = rg_ref[:, 0:DIMS]
    pl48 = jnp.concatenate(
        [g2[:, k * DW:k * DW + DIMS] for k in range(K)], axis=1)
    pl48 = pl48 - jnp.concatenate([rgc] * K, axis=1)

    x0 = _elu(jnp.dot(pl48, wc48_ref[...],
                      preferred_element_type=jnp.float32) + bc_ref[...])
    x1 = _elu(jnp.dot(x0, wdc1b_ref[...],
                      preferred_element_type=jnp.float32) + bdc1_ref[...])
    x2 = jnp.dot(x1, wdc2b_ref[...],
                 preferred_element_type=jnp.float32) + bdc2_ref[...]
    x5tt = x2.reshape(TT, K, K)

    fx = x5tt[:, :, 0:1] * fc3[:, 0:1, :]
    for j in range(1, K):
        fx = fx + x5tt[:, :, j:j + 1] * fc3[:, j:j + 1, :]

    dws = []
    for m in range(DM):
        t = fx * wsdt_ref[m][None]
        dws.append(jnp.sum(t, axis=1, keepdims=True).reshape(TT, C_CAT))
    dwb = jnp.concatenate(dws, axis=1) + bsd_ref[...]
    out = _elu(jnp.dot(dwb, wsp_ref[...],
                       preferred_element_type=jnp.float32) + bsp_ref[...])
    out_ref[...] = out


def _tail_call(g3, g2, rg, wd1, bd1, wd2, bd2, wc48, bc, wdc1b, bdc1,
               wdc2b, bdc2, wsdt, bsd_r, wsp_r, bsp):
    grid = (B * P // TT,)
    full = lambda a: pl.BlockSpec(a.shape, lambda i: (0,) * a.ndim)
    return pl.pallas_call(
        _tail_body,
        grid=grid,
        in_specs=[
            pl.BlockSpec((TT, K, DW), lambda i: (i, 0, 0)),
            pl.BlockSpec((TT, K * DW), lambda i: (i, 0)),
            pl.BlockSpec((TT, DW), lambda i: (i, 0)),
            full(wd1), full(bd1), full(wd2), full(bd2),
            full(wc48), full(bc), full(wdc1b), full(bdc1),
            full(wdc2b), full(bdc2), full(wsdt), full(bsd_r),
            full(wsp_r), full(bsp),
        ],
        out_specs=pl.BlockSpec((TT, C_OUT), lambda i: (i, 0)),
        out_shape=jax.ShapeDtypeStruct((B * P, C_OUT), jnp.float32),
    )(g3, g2, rg, wd1, bd1, wd2, bd2, wc48, bc, wdc1b, bdc1, wdc2b, bdc2,
      wsdt, bsd_r, wsp_r, bsp)


def kernel(pts, fts, W_dense, b_dense, W_d1, b_d1, W_d2, b_d2, W_conv,
           b_conv, W_dc1, b_dc1, W_dc2, b_dc2, W_sd, b_sd, W_sp, b_sp):
    px = pts[:, :, 0]
    py = pts[:, :, 1]
    pz = pts[:, :, 2]

    rx, ry, rz, cg = _fps_call(px, py, pz)

    table = _table_call(pts.reshape(B * N, DIMS), fts.reshape(B * N, C_IN),
                        W_dense, b_dense.reshape(1, C_HALF))

    idx3 = _knn_call(rx, ry, rz, px, py, pz)
    idx_all = jnp.concatenate([idx3.reshape(-1), cg.reshape(-1)])

    rows = _sc_gather_call(table, idx_all)
    g3 = rows[:B * P * K].reshape(B * P, K, DW)
    g2 = rows[:B * P * K].reshape(B * P, K * DW)
    rg = rows[B * P * K:]

    wc48 = W_conv.transpose(2, 1, 0).reshape(K * DIMS, K * K)
    delta = jnp.eye(K, dtype=jnp.float32)
    wdc1b = jnp.einsum('gmk,hg->khgm', W_dc1, delta).reshape(K * K, K * K)
    wdc2b = jnp.einsum('gmk,hg->khgm', W_dc2, delta).reshape(K * K, K * K)
    wsdt = W_sd.transpose(1, 2, 0)
    bsd_r = b_sd.reshape(C_CAT, DM).T.reshape(1, DM * C_CAT)
    wsp_r = W_sp.reshape(C_CAT, DM, C_OUT).transpose(1, 0, 2).reshape(
        DM * C_CAT, C_OUT)

    fts_p = _tail_call(g3, g2, rg, W_d1, b_d1.reshape(1, C_MID), W_d2,
                       b_d2.reshape(1, C_MID), wc48, b_conv.reshape(1, K * K),
                       wdc1b, b_dc1.reshape(1, K * K),
                       wdc2b, b_dc2.reshape(1, K * K), wsdt, bsd_r, wsp_r,
                       b_sp.reshape(1, C_OUT))

    rep_pts = jnp.stack([rx, ry, rz], axis=-1)
    return rep_pts, fts_p.reshape(B, P, C_OUT)


# speedup vs baseline: 1.8415x; 1.0050x over previous
import functools

import jax
import jax.numpy as jnp
from jax import lax
from jax.experimental import pallas as pl
from jax.experimental.pallas import tpu as pltpu
from jax.experimental.pallas import tpu_sc as plsc

B, N, DIMS = 8, 4096, 3
C_IN, C_OUT, K, P = 64, 128, 16, 1024
C_MID = 32
C_HALF = 64
C_CAT = 96
DM = 2
DW = 128

Q = 256
TT = 256
NROWS = B * P * K + B * P


def _elu(x):
    return jnp.where(x > 0, x, jnp.exp(x) - 1.0)


def _col_from_row(row, n):
    r2 = jnp.broadcast_to(row, (n, n))
    ii = lax.broadcasted_iota(jnp.int32, (n, n), 0)
    jj = lax.broadcasted_iota(jnp.int32, (n, n), 1)
    return jnp.sum(jnp.where(ii == jj, r2, jnp.zeros_like(r2)), axis=1,
                   keepdims=True)


def _fps_body(px_ref, py_ref, pz_ref, rx_ref, ry_ref, rz_ref, cg_ref):
    px = px_ref[...]
    py = py_ref[...]
    pz = pz_ref[...]
    iota_n = lax.broadcasted_iota(jnp.int32, (B, N), 1)
    iota_p = lax.broadcasted_iota(jnp.int32, (B, P), 1)
    boff = lax.broadcasted_iota(jnp.int32, (B, 1), 0) * N
    ninf = jnp.float32(-jnp.inf)

    def body(i, carry):
        dist, far, cx, cy, cz, rx, ry, rz, cg = carry
        ohp = (iota_p == i)
        rx = rx + jnp.where(ohp, cx, 0.0)
        ry = ry + jnp.where(ohp, cy, 0.0)
        rz = rz + jnp.where(ohp, cz, 0.0)
        cg = cg + jnp.where(ohp, far + boff, 0)
        dx = px - cx
        dy = py - cy
        dz = pz - cz
        d = (dx * dx + dy * dy) + dz * dz
        dist = jnp.minimum(dist, d)
        mx = jnp.max(dist, axis=1, keepdims=True)
        sel = dist == mx
        far = jnp.min(jnp.where(sel, iota_n, N), axis=1, keepdims=True)
        cx = jnp.max(jnp.where(sel, px, ninf), axis=1, keepdims=True)
        cy = jnp.max(jnp.where(sel, py, ninf), axis=1, keepdims=True)
        cz = jnp.max(jnp.where(sel, pz, ninf), axis=1, keepdims=True)
        return dist, far, cx, cy, cz, rx, ry, rz, cg

    dist0 = jnp.full((B, N), 1e10, jnp.float32)
    far0 = jnp.zeros((B, 1), jnp.int32)
    z = jnp.zeros((B, P), jnp.float32)
    zi = jnp.zeros((B, P), jnp.int32)
    st = lax.fori_loop(0, P, body,
                       (dist0, far0, px[:, 0:1], py[:, 0:1], pz[:, 0:1],
                        z, z, z, zi))
    _, _, _, _, _, rx, ry, rz, cg = st
    rx_ref[...] = rx
    ry_ref[...] = ry
    rz_ref[...] = rz
    cg_ref[...] = cg


def _fps_call(px, py, pz):
    out = [jax.ShapeDtypeStruct((B, P), jnp.float32)] * 3 + [
        jax.ShapeDtypeStruct((B, P), jnp.int32)]
    return pl.pallas_call(_fps_body, out_shape=tuple(out))(px, py, pz)


def _table_body(pts_ref, fts_ref, w_ref, b_ref, out_ref):
    t = _elu(jnp.dot(fts_ref[...], w_ref[...],
                     preferred_element_type=jnp.float32) + b_ref[...])
    pad = jnp.zeros((pts_ref.shape[0], DW - DIMS - C_HALF), jnp.float32)
    out_ref[...] = jnp.concatenate([pts_ref[...], t, pad], axis=1)


def _table_call(pts_f, fts_f, w, b):
    tb = 2048
    grid = (B * N // tb,)
    return pl.pallas_call(
        _table_body,
        grid=grid,
        in_specs=[
            pl.BlockSpec((tb, DIMS), lambda i: (i, 0)),
            pl.BlockSpec((tb, C_IN), lambda i: (i, 0)),
            pl.BlockSpec((C_IN, C_HALF), lambda i: (0, 0)),
            pl.BlockSpec((1, C_HALF), lambda i: (0, 0)),
        ],
        out_specs=pl.BlockSpec((tb, DW), lambda i: (i, 0)),
        out_shape=jax.ShapeDtypeStruct((B * N, DW), jnp.float32),
    )(pts_f, fts_f, w, b)


def _knn_body(rx_ref, ry_ref, rz_ref, px_ref, py_ref, pz_ref, out_ref):
    b = pl.program_id(0)
    qx = _col_from_row(rx_ref[...].reshape(1, Q), Q)
    qy = _col_from_row(ry_ref[...].reshape(1, Q), Q)
    qz = _col_from_row(rz_ref[...].reshape(1, Q), Q)
    dx = qx - px_ref[...].reshape(1, N)
    dy = qy - py_ref[...].reshape(1, N)
    dz = qz - pz_ref[...].reshape(1, N)
    d2 = (dx * dx + dy * dy) + dz * dz
    iota_n = lax.broadcasted_iota(jnp.int32, (Q, N), 1)
    cols = []
    for _ in range(K):
        m = jnp.min(d2, axis=1, keepdims=True)
        sel = jnp.min(jnp.where(d2 == m, iota_n, N), axis=1, keepdims=True)
        cols.append(sel)
        d2 = jnp.where(iota_n == sel, jnp.inf, d2)
    idx = jnp.concatenate(cols, axis=1) + b * N
    out_ref[0] = idx


def _knn_call(rx, ry, rz, px, py, pz):
    grid = (B, P // Q)
    nq = P // Q
    r3 = lambda a: a.reshape(B * nq, 1, Q)
    p3 = lambda a: a.reshape(B, 1, N)
    rspec = pl.BlockSpec((1, 1, Q), lambda b, q: (b * nq + q, 0, 0))
    pspec = pl.BlockSpec((1, 1, N), lambda b, q: (b, 0, 0))
    return pl.pallas_call(
        _knn_body,
        grid=grid,
        in_specs=[rspec, rspec, rspec, pspec, pspec, pspec],
        out_specs=pl.BlockSpec((1, Q, K), lambda b, q: (b, q, 0)),
        out_shape=jax.ShapeDtypeStruct((B, P, K), jnp.int32),
    )(r3(rx), r3(ry), r3(rz), p3(px), p3(py), p3(pz))


NW = 32
RPW = NROWS // NW
CH = 128
NCH = RPW // CH


def _sc_gather_body(table_hbm, idx_hbm, out_hbm, idx_v, buf, sem):
    wid = lax.axis_index("s") * 2 + lax.axis_index("c")
    base = wid * RPW
    pltpu.sync_copy(idx_hbm.at[pl.ds(base, RPW)], idx_v)

    def chunk(g, carry):
        off = g * CH
        pltpu.async_copy(table_hbm.at[idx_v.at[pl.ds(off, CH)]], buf,
                         sem).wait()
        pltpu.sync_copy(buf, out_hbm.at[pl.ds(base + off, CH)])
        return carry

    lax.fori_loop(0, NCH, chunk, 0)


def _sc_gather_call(table, idx_all):
    mesh = plsc.VectorSubcoreMesh(core_axis_name="c", subcore_axis_name="s")
    f = pl.kernel(
        _sc_gather_body,
        out_type=jax.ShapeDtypeStruct((NROWS, DW), jnp.float32),
        mesh=mesh,
        scratch_types=[
            pltpu.VMEM((RPW,), jnp.int32),
            pltpu.VMEM((CH, DW), jnp.float32),
            pltpu.SemaphoreType.DMA,
        ],
    )
    return f(table, idx_all)


def _tail_body(g2_ref, rg_ref, wd1b_ref, bd1t_ref, wd2b_ref, bd2t_ref,
               wc48_ref, bc_ref, wdc1b_ref, bdc1_ref, wdc2b_ref,
               bdc2_ref, wsdt_ref, bsd_ref, wsp_ref, bsp_ref, out_ref):
    g2 = g2_ref[...]

    rgc = rg_ref[:, 0:DIMS]
    pl48 = jnp.concatenate(
        [g2[:, k * DW:k * DW + DIMS] for k in range(K)], axis=1)
    pl48 = pl48 - jnp.concatenate([rgc] * K, axis=1)

    l1 = _elu(jnp.dot(pl48, wd1b_ref[...],
                      preferred_element_type=jnp.float32) + bd1t_ref[...])
    l2 = _elu(jnp.dot(l1, wd2b_ref[...],
                      preferred_element_type=jnp.float32) + bd2t_ref[...])
    fcr = [jnp.concatenate(
        [l2[:, j * C_MID:(j + 1) * C_MID],
         g2[:, j * DW + DIMS:j * DW + DIMS + C_HALF]],
        axis=1).reshape(TT, 1, C_CAT) for j in range(K)]

    x0 = _elu(jnp.dot(pl48, wc48_ref[...],
                      preferred_element_type=jnp.float32) + bc_ref[...])
    x1 = _elu(jnp.dot(x0, wdc1b_ref[...],
                      preferred_element_type=jnp.float32) + bdc1_ref[...])
    x2 = jnp.dot(x1, wdc2b_ref[...],
                 preferred_element_type=jnp.float32) + bdc2_ref[...]
    x5tt = x2.reshape(TT, K, K)

    fx = x5tt[:, :, 0:1] * fcr[0]
    for j in range(1, K):
        fx = fx + x5tt[:, :, j:j + 1] * fcr[j]

    dws = []
    for m in range(DM):
        t = fx * wsdt_ref[m][None]
        dws.append(jnp.sum(t, axis=1, keepdims=True).reshape(TT, C_CAT))
    dwb = jnp.concatenate(dws, axis=1) + bsd_ref[...]
    out = _elu(jnp.dot(dwb, wsp_ref[...],
                       preferred_element_type=jnp.float32) + bsp_ref[...])
    out_ref[...] = out


def _tail_call(g2, rg, wd1b, bd1t, wd2b, bd2t, wc48, bc, wdc1b, bdc1,
               wdc2b, bdc2, wsdt, bsd_r, wsp_r, bsp):
    grid = (B * P // TT,)
    full = lambda a: pl.BlockSpec(a.shape, lambda i: (0,) * a.ndim)
    return pl.pallas_call(
        _tail_body,
        grid=grid,
        in_specs=[
            pl.BlockSpec((TT, K * DW), lambda i: (i, 0)),
            pl.BlockSpec((TT, DW), lambda i: (i, 0)),
            full(wd1b), full(bd1t), full(wd2b), full(bd2t),
            full(wc48), full(bc), full(wdc1b), full(bdc1),
            full(wdc2b), full(bdc2), full(wsdt), full(bsd_r),
            full(wsp_r), full(bsp),
        ],
        out_specs=pl.BlockSpec((TT, C_OUT), lambda i: (i, 0)),
        out_shape=jax.ShapeDtypeStruct((B * P, C_OUT), jnp.float32),
    )(g2, rg, wd1b, bd1t, wd2b, bd2t, wc48, bc, wdc1b, bdc1, wdc2b, bdc2,
      wsdt, bsd_r, wsp_r, bsp)


def kernel(pts, fts, W_dense, b_dense, W_d1, b_d1, W_d2, b_d2, W_conv,
           b_conv, W_dc1, b_dc1, W_dc2, b_dc2, W_sd, b_sd, W_sp, b_sp):
    px = pts[:, :, 0]
    py = pts[:, :, 1]
    pz = pts[:, :, 2]

    rx, ry, rz, cg = _fps_call(px, py, pz)

    table = _table_call(pts.reshape(B * N, DIMS), fts.reshape(B * N, C_IN),
                        W_dense, b_dense.reshape(1, C_HALF))

    idx3 = _knn_call(rx, ry, rz, px, py, pz)
    idx_all = jnp.concatenate([idx3.reshape(-1), cg.reshape(-1)])

    rows = _sc_gather_call(table, idx_all)
    g2 = rows[:B * P * K].reshape(B * P, K * DW)
    rg = rows[B * P * K:]

    wc48 = W_conv.transpose(2, 1, 0).reshape(K * DIMS, K * K)
    delta = jnp.eye(K, dtype=jnp.float32)
    wd1b = jnp.einsum('kh,cm->kchm', delta, W_d1).reshape(
        K * DIMS, K * C_MID)
    bd1t = jnp.tile(b_d1, K).reshape(1, K * C_MID)
    wd2b = jnp.einsum('kh,mn->kmhn', delta, W_d2).reshape(
        K * C_MID, K * C_MID)
    bd2t = jnp.tile(b_d2, K).reshape(1, K * C_MID)
    wdc1b = jnp.einsum('gmk,hg->khgm', W_dc1, delta).reshape(K * K, K * K)
    wdc2b = jnp.einsum('gmk,hg->khgm', W_dc2, delta).reshape(K * K, K * K)
    wsdt = W_sd.transpose(1, 2, 0)
    bsd_r = b_sd.reshape(C_CAT, DM).T.reshape(1, DM * C_CAT)
    wsp_r = W_sp.reshape(C_CAT, DM, C_OUT).transpose(1, 0, 2).reshape(
        DM * C_CAT, C_OUT)

    fts_p = _tail_call(g2, rg, wd1b, bd1t, wd2b, bd2t,
                       wc48, b_conv.reshape(1, K * K),
                       wdc1b, b_dc1.reshape(1, K * K),
                       wdc2b, b_dc2.reshape(1, K * K), wsdt, bsd_r, wsp_r,
                       b_sp.reshape(1, C_OUT))

    rep_pts = jnp.stack([rx, ry, rz], axis=-1)
    return rep_pts, fts_p.reshape(B, P, C_OUT)


# speedup vs baseline: 2.1308x; 1.1571x over previous
import jax
import jax.numpy as jnp
from jax import lax
from jax.experimental import pallas as pl
from jax.experimental.pallas import tpu as pltpu
from jax.experimental.pallas import tpu_sc as plsc

B, N, DIMS = 8, 4096, 3
C_IN, C_OUT, K, P = 64, 128, 16, 1024
C_MID = 32
C_HALF = 64
C_CAT = 96
DM = 2
DW = 128

Q = 256
TT = 256
NROWS = B * P * K + B * P


def _elu(x):
    return jnp.where(x > 0, x, jnp.exp(x) - 1.0)


def _col_from_row(row, n):
    r2 = jnp.broadcast_to(row, (n, n))
    ii = lax.broadcasted_iota(jnp.int32, (n, n), 0)
    jj = lax.broadcasted_iota(jnp.int32, (n, n), 1)
    return jnp.sum(jnp.where(ii == jj, r2, jnp.zeros_like(r2)), axis=1,
                   keepdims=True)


def _fps_body(px_ref, py_ref, pz_ref, rx_ref, ry_ref, rz_ref, cg_ref):
    px = px_ref[...]
    py = py_ref[...]
    pz = pz_ref[...]
    iota_n = lax.broadcasted_iota(jnp.int32, (B, N), 1)
    iota_p = lax.broadcasted_iota(jnp.int32, (B, P), 1)
    boff = lax.broadcasted_iota(jnp.int32, (B, 1), 0) * N
    ninf = jnp.float32(-jnp.inf)

    def body(i, carry):
        dist, far, cx, cy, cz, rx, ry, rz, cg = carry
        ohp = (iota_p == i)
        rx = rx + jnp.where(ohp, cx, 0.0)
        ry = ry + jnp.where(ohp, cy, 0.0)
        rz = rz + jnp.where(ohp, cz, 0.0)
        cg = cg + jnp.where(ohp, far + boff, 0)
        dx = px - cx
        dy = py - cy
        dz = pz - cz
        d = (dx * dx + dy * dy) + dz * dz
        dist = jnp.minimum(dist, d)
        mx = jnp.max(dist, axis=1, keepdims=True)
        sel = dist == mx
        far = jnp.min(jnp.where(sel, iota_n, N), axis=1, keepdims=True)
        one = iota_n == far
        cx = jnp.max(jnp.where(one, px, ninf), axis=1, keepdims=True)
        cy = jnp.max(jnp.where(one, py, ninf), axis=1, keepdims=True)
        cz = jnp.max(jnp.where(one, pz, ninf), axis=1, keepdims=True)
        return dist, far, cx, cy, cz, rx, ry, rz, cg

    dist0 = jnp.full((B, N), 1e10, jnp.float32)
    far0 = jnp.zeros((B, 1), jnp.int32)
    z = jnp.zeros((B, P), jnp.float32)
    zi = jnp.zeros((B, P), jnp.int32)
    st = lax.fori_loop(0, P, body,
                       (dist0, far0, px[:, 0:1], py[:, 0:1], pz[:, 0:1],
                        z, z, z, zi))
    _, _, _, _, _, rx, ry, rz, cg = st
    rx_ref[...] = rx
    ry_ref[...] = ry
    rz_ref[...] = rz
    cg_ref[...] = cg


def _fps_call(px, py, pz):
    out = [jax.ShapeDtypeStruct((B, P), jnp.float32)] * 3 + [
        jax.ShapeDtypeStruct((B, P), jnp.int32)]
    return pl.pallas_call(_fps_body, out_shape=tuple(out))(px, py, pz)


def _table_body(pts_ref, fts_ref, w_ref, b_ref, out_ref):
    t = _elu(jnp.dot(fts_ref[...], w_ref[...],
                     preferred_element_type=jnp.float32) + b_ref[...])
    pad = jnp.zeros((pts_ref.shape[0], DW - DIMS - C_HALF), jnp.float32)
    out_ref[...] = jnp.concatenate([pts_ref[...], t, pad], axis=1)


def _table_call(pts_f, fts_f, w, b):
    tb = 2048
    grid = (B * N // tb,)
    return pl.pallas_call(
        _table_body,
        grid=grid,
        in_specs=[
            pl.BlockSpec((tb, DIMS), lambda i: (i, 0)),
            pl.BlockSpec((tb, C_IN), lambda i: (i, 0)),
            pl.BlockSpec((C_IN, C_HALF), lambda i: (0, 0)),
            pl.BlockSpec((1, C_HALF), lambda i: (0, 0)),
        ],
        out_specs=pl.BlockSpec((tb, DW), lambda i: (i, 0)),
        out_shape=jax.ShapeDtypeStruct((B * N, DW), jnp.float32),
    )(pts_f, fts_f, w, b)


def _knn_body(rx_ref, ry_ref, rz_ref, px_ref, py_ref, pz_ref, out_ref):
    b = pl.program_id(0)
    qx = _col_from_row(rx_ref[...].reshape(1, Q), Q)
    qy = _col_from_row(ry_ref[...].reshape(1, Q), Q)
    qz = _col_from_row(rz_ref[...].reshape(1, Q), Q)
    dx = qx - px_ref[...].reshape(1, N)
    dy = qy - py_ref[...].reshape(1, N)
    dz = qz - pz_ref[...].reshape(1, N)
    d2 = (dx * dx + dy * dy) + dz * dz
    iota_n = lax.broadcasted_iota(jnp.int32, (Q, N), 1)
    cols = []
    for _ in range(K):
        m = jnp.min(d2, axis=1, keepdims=True)
        sel = jnp.min(jnp.where(d2 == m, iota_n, N), axis=1, keepdims=True)
        cols.append(sel)
        d2 = jnp.where(iota_n == sel, jnp.inf, d2)
    idx = jnp.concatenate(cols, axis=1) + b * N
    out_ref[0] = idx


def _knn_call(rx, ry, rz, px, py, pz):
    grid = (B, P // Q)
    nq = P // Q
    r3 = lambda a: a.reshape(B * nq, 1, Q)
    p3 = lambda a: a.reshape(B, 1, N)
    rspec = pl.BlockSpec((1, 1, Q), lambda b, q: (b * nq + q, 0, 0))
    pspec = pl.BlockSpec((1, 1, N), lambda b, q: (b, 0, 0))
    return pl.pallas_call(
        _knn_body,
        grid=grid,
        in_specs=[rspec, rspec, rspec, pspec, pspec, pspec],
        out_specs=pl.BlockSpec((1, Q, K), lambda b, q: (b, q, 0)),
        out_shape=jax.ShapeDtypeStruct((B, P, K), jnp.int32),
    )(r3(rx), r3(ry), r3(rz), p3(px), p3(py), p3(pz))


NW = 32
RPW = NROWS // NW
CH = 128
NCH = RPW // CH


def _sc_gather_body(table_hbm, idx_hbm, out_hbm, idx_v, buf, sem):
    wid = lax.axis_index("s") * 2 + lax.axis_index("c")
    base = wid * RPW
    pltpu.sync_copy(idx_hbm.at[pl.ds(base, RPW)], idx_v)

    def chunk(g, carry):
        off = g * CH
        pltpu.async_copy(table_hbm.at[idx_v.at[pl.ds(off, CH)]], buf,
                         sem).wait()
        pltpu.sync_copy(buf, out_hbm.at[pl.ds(base + off, CH)])
        return carry

    lax.fori_loop(0, NCH, chunk, 0)


def _sc_gather_call(table, idx_all):
    mesh = plsc.VectorSubcoreMesh(core_axis_name="c", subcore_axis_name="s")
    f = pl.kernel(
        _sc_gather_body,
        out_type=jax.ShapeDtypeStruct((NROWS, DW), jnp.float32),
        mesh=mesh,
        scratch_types=[
            pltpu.VMEM((RPW,), jnp.int32),
            pltpu.VMEM((CH, DW), jnp.float32),
            pltpu.SemaphoreType.DMA,
        ],
    )
    return f(table, idx_all)


def _tail_body(g2_ref, rg_ref, wd1b_ref, bd1t_ref, wd2b_ref, bd2t_ref,
               wc48_ref, bc_ref, wdc1b_ref, bdc1_ref, wdc2bp_ref,
               bdc2p_ref, wsd2_ref, bsd_ref, wsp_ref, bsp_ref, out_ref):
    g2 = g2_ref[...]

    rgc = rg_ref[:, 0:DIMS]
    pl48 = jnp.concatenate(
        [g2[:, k * DW:k * DW + DIMS] for k in range(K)], axis=1)
    pl48 = pl48 - jnp.concatenate([rgc] * K, axis=1)

    l1 = _elu(jnp.dot(pl48, wd1b_ref[...],
                      preferred_element_type=jnp.float32) + bd1t_ref[...])
    l2 = _elu(jnp.dot(l1, wd2b_ref[...],
                      preferred_element_type=jnp.float32) + bd2t_ref[...])

    x0 = _elu(jnp.dot(pl48, wc48_ref[...],
                      preferred_element_type=jnp.float32) + bc_ref[...])
    x1 = _elu(jnp.dot(x0, wdc1b_ref[...],
                      preferred_element_type=jnp.float32) + bdc1_ref[...])
    x2p = jnp.dot(x1, wdc2bp_ref[...],
                  preferred_element_type=jnp.float32) + bdc2p_ref[...]

    acc = None
    for j in range(K):
        x5j = x2p[:, j * K:(j + 1) * K]
        sj = jnp.dot(x5j, wsd2_ref[...],
                     preferred_element_type=jnp.float32)
        fcj = jnp.concatenate(
            [l2[:, j * C_MID:(j + 1) * C_MID],
             g2[:, j * DW + DIMS:j * DW + DIMS + C_HALF]], axis=1)
        fce = jnp.concatenate([fcj] * DM, axis=1)
        t = fce * sj
        acc = t if acc is None else acc + t

    dwb = acc + bsd_ref[...]
    out = _elu(jnp.dot(dwb, wsp_ref[...],
                       preferred_element_type=jnp.float32) + bsp_ref[...])
    out_ref[...] = out


def _tail_call(g2, rg, wd1b, bd1t, wd2b, bd2t, wc48, bc, wdc1b, bdc1,
               wdc2bp, bdc2p, wsd2, bsd_r, wsp_r, bsp):
    grid = (B * P // TT,)
    full = lambda a: pl.BlockSpec(a.shape, lambda i: (0,) * a.ndim)
    return pl.pallas_call(
        _tail_body,
        grid=grid,
        in_specs=[
            pl.BlockSpec((TT, K * DW), lambda i: (i, 0)),
            pl.BlockSpec((TT, DW), lambda i: (i, 0)),
            full(wd1b), full(bd1t), full(wd2b), full(bd2t),
            full(wc48), full(bc), full(wdc1b), full(bdc1),
            full(wdc2bp), full(bdc2p), full(wsd2), full(bsd_r),
            full(wsp_r), full(bsp),
        ],
        out_specs=pl.BlockSpec((TT, C_OUT), lambda i: (i, 0)),
        out_shape=jax.ShapeDtypeStruct((B * P, C_OUT), jnp.float32),
    )(g2, rg, wd1b, bd1t, wd2b, bd2t, wc48, bc, wdc1b, bdc1, wdc2bp,
      bdc2p, wsd2, bsd_r, wsp_r, bsp)


def kernel(pts, fts, W_dense, b_dense, W_d1, b_d1, W_d2, b_d2, W_conv,
           b_conv, W_dc1, b_dc1, W_dc2, b_dc2, W_sd, b_sd, W_sp, b_sp):
    px = pts[:, :, 0]
    py = pts[:, :, 1]
    pz = pts[:, :, 2]

    rx, ry, rz, cg = _fps_call(px, py, pz)

    table = _table_call(pts.reshape(B * N, DIMS), fts.reshape(B * N, C_IN),
                        W_dense, b_dense.reshape(1, C_HALF))

    idx3 = _knn_call(rx, ry, rz, px, py, pz)
    idx_all = jnp.concatenate([idx3.reshape(-1), cg.reshape(-1)])

    rows = _sc_gather_call(table, idx_all)
    g2 = rows[:B * P * K].reshape(B * P, K * DW)
    rg = rows[B * P * K:]

    wc48 = W_conv.transpose(2, 1, 0).reshape(K * DIMS, K * K)
    delta = jnp.eye(K, dtype=jnp.float32)
    wd1b = jnp.einsum('kh,cm->kchm', delta, W_d1).reshape(
        K * DIMS, K * C_MID)
    bd1t = jnp.tile(b_d1, K).reshape(1, K * C_MID)
    wd2b = jnp.einsum('kh,mn->kmhn', delta, W_d2).reshape(
        K * C_MID, K * C_MID)
    bd2t = jnp.tile(b_d2, K).reshape(1, K * C_MID)
    wdc1b = jnp.einsum('gmk,hg->khgm', W_dc1, delta).reshape(K * K, K * K)
    wdc2b = jnp.einsum('gmk,hg->khgm', W_dc2, delta).reshape(K * K, K * K)
    wdc2bp = wdc2b.reshape(K * K, K, K).transpose(0, 2, 1).reshape(
        K * K, K * K)
    bdc2p = b_dc2.reshape(K, K).T.reshape(1, K * K)
    wsd2 = W_sd.transpose(2, 1, 0).reshape(K, DM * C_CAT)
    bsd_r = b_sd.reshape(C_CAT, DM).T.reshape(1, DM * C_CAT)
    wsp_r = W_sp.reshape(C_CAT, DM, C_OUT).transpose(1, 0, 2).reshape(
        DM * C_CAT, C_OUT)

    fts_p = _tail_call(g2, rg, wd1b, bd1t, wd2b, bd2t,
                       wc48, b_conv.reshape(1, K * K),
                       wdc1b, b_dc1.reshape(1, K * K),
                       wdc2bp, bdc2p, wsd2, bsd_r, wsp_r,
                       b_sp.reshape(1, C_OUT))

    rep_pts = jnp.stack([rx, ry, rz], axis=-1)
    return rep_pts, fts_p.reshape(B, P, C_OUT)
